# baseline (speedup 1.0000x reference)
import functools

import jax
import jax.numpy as jnp
from jax import lax
from jax.experimental import pallas as pl
from jax.experimental.pallas import tpu as pltpu
from jax.experimental.pallas import tpu_sc as plsc

_NEG_INF = float(-1e37)
_IDX_CHUNK = 128


def _chunks(n):
    out, off = [], 0
    while off < n:
        sz = min(_IDX_CHUNK, n - off)
        out.append((off, sz))
        off += sz
    return out


@functools.partial(jax.jit, static_argnames=("naug",))
def _sc_gather(weight2, bias128, aug_idx, label_idx, *, naug):
    d2 = weight2.shape[1]
    nw, n1 = aug_idx.shape
    n2 = label_idx.shape[1]
    b = nw * n2
    info = plsc.get_sparse_core_info()
    nc = info.num_cores
    ch1, ch2 = _chunks(n1), _chunks(n2)

    mesh = plsc.VectorSubcoreMesh(core_axis_name="c", subcore_axis_name="s")

    @functools.partial(
        pl.kernel,
        out_type=(
            jax.ShapeDtypeStruct((naug, d2), jnp.float32),
            jax.ShapeDtypeStruct((nw, n1), jnp.float32),
            jax.ShapeDtypeStruct((b, d2), jnp.float32),
            jax.ShapeDtypeStruct((nw, n2), jnp.float32),
        ),
        mesh=mesh,
        compiler_params=pltpu.CompilerParams(needs_layout_passes=False),
        scratch_types=(
            pltpu.VMEM((n1,), jnp.int32),
            pltpu.VMEM((n1,), jnp.int32),
            pltpu.VMEM((n1,), jnp.int32),
            pltpu.VMEM((n1, d2), jnp.float32),
            pltpu.VMEM((n1, 128), jnp.float32),
            pltpu.VMEM((n1,), jnp.float32),
            pltpu.VMEM((n2,), jnp.int32),
            pltpu.VMEM((n2,), jnp.int32),
            pltpu.VMEM((n2,), jnp.int32),
            pltpu.VMEM((n2, d2), jnp.float32),
            pltpu.VMEM((n2, 128), jnp.float32),
            pltpu.VMEM((n2,), jnp.float32),
            pltpu.SemaphoreType.DMA,
        ),
    )
    def gather(w_hbm, b128_hbm, aidx_hbm, lidx_hbm,
               wa_out, ba_out, tw_out, tb_out,
               idx1, idx1h, idx1b, rows1, b128_1, bsel1,
               idx2, idx2h, idx2b, rows2, b128_2, bsel2, sem):
        iota16 = lax.iota(jnp.int32, 16)
        wid = lax.axis_index("s") * nc + lax.axis_index("c")
        base1 = wid * n1
        base2 = wid * n2
        pltpu.sync_copy(aidx_hbm.at[wid], idx1)
        pltpu.sync_copy(lidx_hbm.at[wid], idx2)
        for idx, idxh, idxb, n in ((idx1, idx1h, idx1b, n1),
                                   (idx2, idx2h, idx2b, n2)):
            for g in range(n // 16):
                sl = pl.ds(16 * g, 16)
                val = idx[sl]
                idxh[sl] = lax.shift_right_logical(val, 1)
                idxb[sl] = lax.shift_right_logical(val, 7)
        copies = []
        for off, sz in ch1:
            copies.append(pltpu.async_copy(
                w_hbm.at[idx1h.at[pl.ds(off, sz)]], rows1.at[pl.ds(off, sz)], sem))
            copies.append(pltpu.async_copy(
                b128_hbm.at[idx1b.at[pl.ds(off, sz)]], b128_1.at[pl.ds(off, sz)], sem))
        for off, sz in ch2:
            copies.append(pltpu.async_copy(
                w_hbm.at[idx2h.at[pl.ds(off, sz)]], rows2.at[pl.ds(off, sz)], sem))
            copies.append(pltpu.async_copy(
                b128_hbm.at[idx2b.at[pl.ds(off, sz)]], b128_2.at[pl.ds(off, sz)], sem))
        for c in copies:
            c.wait()
        for idx, b128v, bsel, n in ((idx1, b128_1, bsel1, n1),
                                    (idx2, b128_2, bsel2, n2)):
            for g in range(n // 16):
                sl = pl.ds(16 * g, 16)
                rows = 16 * g + iota16
                lanes = lax.bitwise_and(idx[sl], 127)
                bsel[sl] = plsc.load_gather(b128v, [rows, lanes])
        pltpu.sync_copy(rows1, wa_out.at[pl.ds(base1, n1)])
        pltpu.sync_copy(bsel1, ba_out.at[wid])
        pltpu.sync_copy(rows2, tw_out.at[pl.ds(base2, n2)])
        pltpu.sync_copy(bsel2, tb_out.at[wid])

    return gather(weight2, bias128, aug_idx, label_idx)


def _tc_body(x_ref, w2_ref, ba_ref, sf_ref, ids_ref, par_ref, lab_ref,
             tw2_ref, lpar_ref, tb_ref, tf_ref, out_ref, *, nout, d):
    x = x_ref[...]
    w2 = w2_ref[...]
    w_sel = jnp.where(par_ref[...] != 0, w2[:, d:], w2[:, :d])
    acc = lax.dot_general(x, w_sel, (((1,), (1,)), ((), ())),
                          preferred_element_type=jnp.float32)
    val = acc + (ba_ref[...] - jnp.log(sf_ref[...]))
    val = jnp.where(lab_ref[...] == ids_ref[...], _NEG_INF, val)
    tw2 = tw2_ref[...]
    tw = jnp.where(lpar_ref[...] != 0, tw2[:, d:], tw2[:, :d])
    t = (jnp.sum(x * tw, axis=1, keepdims=True)
         + tb_ref[...] - jnp.log(tf_ref[...]))
    col0 = lax.broadcasted_iota(jnp.int32, val.shape, 1) == 0
    val = jnp.where(col0, t, val)
    out_ref[...] = val[:, :nout]


def kernel(inputs, labels, weight, bias, sample_ids, true_freq, sample_freq):
    b, d = inputs.shape
    ns = sample_ids.shape[0]
    v = weight.shape[0]
    nout = ns + 1

    labels_i = labels.astype(jnp.int32)
    sids_i = sample_ids.astype(jnp.int32)

    info = plsc.get_sparse_core_info()
    nw = info.num_cores * info.num_subcores
    step = max(16 * nw, 128)
    naug = ((nout + step - 1) // step) * step

    pad = naug - 1 - ns
    aug_idx = jnp.concatenate(
        [jnp.zeros((1,), jnp.int32), sids_i, jnp.zeros((pad,), jnp.int32)])
    aug_ids = jnp.concatenate(
        [jnp.full((1,), -1, jnp.int32), sids_i, jnp.full((pad,), -1, jnp.int32)]
    ).reshape(1, naug)
    aug_sf = jnp.concatenate(
        [jnp.ones((1,), jnp.float32), sample_freq, jnp.ones((pad,), jnp.float32)]
    ).reshape(1, naug)
    aug_par = lax.bitwise_and(aug_idx, 1).reshape(naug, 1)
    lab_par = lax.bitwise_and(labels_i, 1).reshape(b, 1)

    vb = ((v + 127) // 128) * 128
    bias128 = jnp.pad(bias, (0, vb - v)).reshape(vb // 128, 128)
    weight2 = weight.reshape(v // 2, 2 * d)

    w2_aug, b_aug, tw2, tb = _sc_gather(
        weight2, bias128, aug_idx.reshape(nw, naug // nw),
        labels_i.reshape(nw, b // nw), naug=naug)
    b_aug = b_aug.reshape(1, naug)
    tb = tb.reshape(b, 1)

    br = 256
    grid = (b // br,)
    out = pl.pallas_call(
        functools.partial(_tc_body, nout=nout, d=d),
        grid=grid,
        in_specs=[
            pl.BlockSpec((br, d), lambda i: (i, 0)),
            pl.BlockSpec((naug, 2 * d), lambda i: (0, 0)),
            pl.BlockSpec((1, naug), lambda i: (0, 0)),
            pl.BlockSpec((1, naug), lambda i: (0, 0)),
            pl.BlockSpec((1, naug), lambda i: (0, 0)),
            pl.BlockSpec((naug, 1), lambda i: (0, 0)),
            pl.BlockSpec((br, 1), lambda i: (i, 0)),
            pl.BlockSpec((br, 2 * d), lambda i: (i, 0)),
            pl.BlockSpec((br, 1), lambda i: (i, 0)),
            pl.BlockSpec((br, 1), lambda i: (i, 0)),
            pl.BlockSpec((br, 1), lambda i: (i, 0)),
        ],
        out_specs=pl.BlockSpec((br, nout), lambda i: (i, 0)),
        out_shape=jax.ShapeDtypeStruct((b, nout), jnp.float32),
    )(inputs, w2_aug, b_aug, aug_sf, aug_ids, aug_par, labels_i.reshape(b, 1),
      tw2, lab_par, tb, true_freq.reshape(b, 1))
    return out


# speedup vs baseline: 1.1687x; 1.1687x over previous
import functools

import jax
import jax.numpy as jnp
from jax import lax
from jax.experimental import pallas as pl
from jax.experimental.pallas import tpu as pltpu
from jax.experimental.pallas import tpu_sc as plsc

_NEG_INF = float(-1e37)
_IDX_CHUNK = 128


def _chunks(n):
    out, off = [], 0
    while off < n:
        sz = min(_IDX_CHUNK, n - off)
        out.append((off, sz))
        off += sz
    return out


@functools.partial(jax.jit, static_argnames=("naug",))
def _sc_gather(weight, bias16, aug_idx, label_idx, *, naug):
    d = weight.shape[1]
    nw, n1 = aug_idx.shape
    n2 = label_idx.shape[1]
    b = nw * n2
    info = plsc.get_sparse_core_info()
    nc = info.num_cores
    ch1, ch2 = _chunks(n1), _chunks(n2)

    mesh = plsc.VectorSubcoreMesh(core_axis_name="c", subcore_axis_name="s")

    @functools.partial(
        pl.kernel,
        out_type=(
            jax.ShapeDtypeStruct((naug, d), jnp.float32),
            jax.ShapeDtypeStruct((nw, n1), jnp.float32),
            jax.ShapeDtypeStruct((b, d), jnp.float32),
            jax.ShapeDtypeStruct((nw, n2), jnp.float32),
        ),
        mesh=mesh,
        compiler_params=pltpu.CompilerParams(
            use_tc_tiling_on_sc=False, needs_layout_passes=False),
        scratch_types=(
            pltpu.VMEM((n1,), jnp.int32),
            pltpu.VMEM((n1,), jnp.int32),
            pltpu.VMEM((n1,), jnp.int32),
            pltpu.VMEM((n1, d), jnp.float32),
            pltpu.VMEM((n1, 16), jnp.float32),
            pltpu.VMEM((n1,), jnp.float32),
            pltpu.VMEM((n2,), jnp.int32),
            pltpu.VMEM((n2,), jnp.int32),
            pltpu.VMEM((n2,), jnp.int32),
            pltpu.VMEM((n2, d), jnp.float32),
            pltpu.VMEM((n2, 16), jnp.float32),
            pltpu.VMEM((n2,), jnp.float32),
            pltpu.SemaphoreType.DMA,
        ),
    )
    def gather(w_hbm, b16_hbm, aidx_hbm, lidx_hbm,
               wa_out, ba_out, tw_out, tb_out,
               idx1, idx1h, idx1l, rows1, b16_1, bsel1,
               idx2, idx2h, idx2l, rows2, b16_2, bsel2, sem):
        iota16 = lax.iota(jnp.int32, 16)
        wid = lax.axis_index("s") * nc + lax.axis_index("c")
        base1 = wid * n1
        base2 = wid * n2
        pltpu.sync_copy(aidx_hbm.at[wid], idx1)
        pltpu.sync_copy(lidx_hbm.at[wid], idx2)
        for idx, idxh, idxl, n in ((idx1, idx1h, idx1l, n1),
                                   (idx2, idx2h, idx2l, n2)):
            for g in range(n // 16):
                sl = pl.ds(16 * g, 16)
                val = idx[sl]
                idxh[sl] = lax.shift_right_logical(val, 4)
                idxl[sl] = lax.bitwise_and(val, 15)
        copies = []
        for off, sz in ch1:
            copies.append(pltpu.async_copy(
                w_hbm.at[idx1.at[pl.ds(off, sz)]], rows1.at[pl.ds(off, sz)], sem))
            copies.append(pltpu.async_copy(
                b16_hbm.at[idx1h.at[pl.ds(off, sz)]], b16_1.at[pl.ds(off, sz)], sem))
        for off, sz in ch2:
            copies.append(pltpu.async_copy(
                w_hbm.at[idx2.at[pl.ds(off, sz)]], rows2.at[pl.ds(off, sz)], sem))
            copies.append(pltpu.async_copy(
                b16_hbm.at[idx2h.at[pl.ds(off, sz)]], b16_2.at[pl.ds(off, sz)], sem))
        for c in copies:
            c.wait()
        for idxl, b16v, bsel, n in ((idx1l, b16_1, bsel1, n1),
                                    (idx2l, b16_2, bsel2, n2)):
            for g in range(n // 16):
                sl = pl.ds(16 * g, 16)
                rows = 16 * g + iota16
                bsel[sl] = plsc.load_gather(b16v, [rows, idxl[sl]])
        pltpu.sync_copy(rows1, wa_out.at[pl.ds(base1, n1)])
        pltpu.sync_copy(bsel1, ba_out.at[wid])
        pltpu.sync_copy(rows2, tw_out.at[pl.ds(base2, n2)])
        pltpu.sync_copy(bsel2, tb_out.at[wid])

    return gather(weight, bias16, aug_idx, label_idx)


def _tc_body(x_ref, w_ref, ba_ref, sf_ref, ids_ref, lab_ref,
             tw_ref, tb_ref, tf_ref, out_ref):
    x = x_ref[...]
    acc = lax.dot_general(x, w_ref[...], (((1,), (1,)), ((), ())),
                          preferred_element_type=jnp.float32)
    val = acc + (ba_ref[...] - jnp.log(sf_ref[...]))
    val = jnp.where(lab_ref[...] == ids_ref[...], _NEG_INF, val)
    t = (jnp.sum(x * tw_ref[...], axis=1, keepdims=True)
         + tb_ref[...] - jnp.log(tf_ref[...]))
    col0 = lax.broadcasted_iota(jnp.int32, val.shape, 1) == 0
    out_ref[...] = jnp.where(col0, t, val)


def kernel(inputs, labels, weight, bias, sample_ids, true_freq, sample_freq):
    b, d = inputs.shape
    ns = sample_ids.shape[0]
    v = weight.shape[0]
    nout = ns + 1

    labels_i = labels.astype(jnp.int32)
    sids_i = sample_ids.astype(jnp.int32)

    info = plsc.get_sparse_core_info()
    nw = info.num_cores * info.num_subcores
    step = max(16 * nw, 128)
    naug = ((nout + step - 1) // step) * step

    pad = naug - 1 - ns
    aug_idx = jnp.concatenate(
        [jnp.zeros((1,), jnp.int32), sids_i, jnp.zeros((pad,), jnp.int32)])
    aug_ids = jnp.concatenate(
        [jnp.full((1,), -1, jnp.int32), sids_i, jnp.full((pad,), -1, jnp.int32)]
    ).reshape(1, naug)
    aug_sf = jnp.concatenate(
        [jnp.ones((1,), jnp.float32), sample_freq, jnp.ones((pad,), jnp.float32)]
    ).reshape(1, naug)

    w_aug, b_aug, tw, tb = _sc_gather(
        weight, bias.reshape(v // 16, 16), aug_idx.reshape(nw, naug // nw),
        labels_i.reshape(nw, b // nw), naug=naug)
    b_aug = b_aug.reshape(1, naug)
    tb = tb.reshape(b, 1)

    br = 256
    grid = (b // br,)
    out = pl.pallas_call(
        _tc_body,
        grid=grid,
        in_specs=[
            pl.BlockSpec((br, d), lambda i: (i, 0)),
            pl.BlockSpec((naug, d), lambda i: (0, 0)),
            pl.BlockSpec((1, naug), lambda i: (0, 0)),
            pl.BlockSpec((1, naug), lambda i: (0, 0)),
            pl.BlockSpec((1, naug), lambda i: (0, 0)),
            pl.BlockSpec((br, 1), lambda i: (i, 0)),
            pl.BlockSpec((br, d), lambda i: (i, 0)),
            pl.BlockSpec((br, 1), lambda i: (i, 0)),
            pl.BlockSpec((br, 1), lambda i: (i, 0)),
        ],
        out_specs=pl.BlockSpec((br, naug), lambda i: (i, 0)),
        out_shape=jax.ShapeDtypeStruct((b, nout), jnp.float32),
    )(inputs, w_aug, b_aug, aug_sf, aug_ids, labels_i.reshape(b, 1),
      tw, tb, true_freq.reshape(b, 1))
    return out


# speedup vs baseline: 1.3762x; 1.1775x over previous
import functools

import jax
import jax.numpy as jnp
from jax import lax
from jax.experimental import pallas as pl
from jax.experimental.pallas import tpu as pltpu
from jax.experimental.pallas import tpu_sc as plsc

_NEG_INF = float(-1e37)
_IDX_CHUNK = 128
_ROW_GRP = 16


def _chunks(n):
    out, off = [], 0
    while off < n:
        sz = min(_IDX_CHUNK, n - off)
        out.append((off, sz))
        off += sz
    return out


@functools.partial(jax.jit, static_argnames=("naug",))
def _sc_gather(weight, bias128, aug_idx, label_idx, *, naug):
    d = weight.shape[1]
    nw, n1 = aug_idx.shape
    n2 = label_idx.shape[1]
    b = nw * n2
    info = plsc.get_sparse_core_info()
    nc = info.num_cores
    ch1, ch2 = _chunks(n1), _chunks(n2)

    mesh = plsc.VectorSubcoreMesh(core_axis_name="c", subcore_axis_name="s")

    @functools.partial(
        pl.kernel,
        out_type=(
            jax.ShapeDtypeStruct((naug, d), jnp.float32),
            jax.ShapeDtypeStruct((nw, n1), jnp.float32),
            jax.ShapeDtypeStruct((b, d), jnp.float32),
            jax.ShapeDtypeStruct((nw, n2), jnp.float32),
        ),
        mesh=mesh,
        compiler_params=pltpu.CompilerParams(needs_layout_passes=False),
        scratch_types=(
            pltpu.VMEM((n1,), jnp.int32),
            pltpu.VMEM((n1,), jnp.int32),
            pltpu.VMEM((n1, d), jnp.float32),
            pltpu.VMEM((n1, 128), jnp.float32),
            pltpu.VMEM((n1,), jnp.float32),
            pltpu.VMEM((n2,), jnp.int32),
            pltpu.VMEM((n2,), jnp.int32),
            pltpu.VMEM((n2, d), jnp.float32),
            pltpu.VMEM((n2, 128), jnp.float32),
            pltpu.VMEM((n2,), jnp.float32),
            pltpu.SemaphoreType.DMA,
            pltpu.SemaphoreType.DMA,
        ),
    )
    def gather(w_hbm, b128_hbm, aidx_hbm, lidx_hbm,
               wa_out, ba_out, tw_out, tb_out,
               idx1, idx1b, rows1, b128_1, bsel1,
               idx2, idx2b, rows2, b128_2, bsel2, sem, rsem):
        iota16 = lax.iota(jnp.int32, 16)
        wid = lax.axis_index("s") * nc + lax.axis_index("c")
        base1 = wid * n1
        base2 = wid * n2
        pltpu.sync_copy(aidx_hbm.at[wid], idx1)
        pltpu.sync_copy(lidx_hbm.at[wid], idx2)
        for idx, idxb, n in ((idx1, idx1b, n1), (idx2, idx2b, n2)):
            for g in range(n // 16):
                sl = pl.ds(16 * g, 16)
                idxb[sl] = lax.shift_right_logical(idx[sl], 7)
        copies = []
        for off, sz in ch1:
            copies.append(pltpu.async_copy(
                b128_hbm.at[idx1b.at[pl.ds(off, sz)]], b128_1.at[pl.ds(off, sz)], sem))
        for off, sz in ch2:
            copies.append(pltpu.async_copy(
                b128_hbm.at[idx2b.at[pl.ds(off, sz)]], b128_2.at[pl.ds(off, sz)], sem))
        for idx, rows, n in ((idx1, rows1, n1), (idx2, rows2, n2)):
            def grp(i, _, idx=idx, rows=rows):
                vec = idx[pl.ds(i * _ROW_GRP, 16)]
                cps = []
                for j in range(_ROW_GRP):
                    k = i * _ROW_GRP + j
                    cps.append(pltpu.async_copy(
                        w_hbm.at[pl.ds(vec[j], 1)], rows.at[pl.ds(k, 1)], rsem))
                for cp in cps:
                    cp.wait()
                return 0
            lax.fori_loop(0, n // _ROW_GRP, grp, 0)
        for c in copies:
            c.wait()
        for idx, b128v, bsel, n in ((idx1, b128_1, bsel1, n1),
                                    (idx2, b128_2, bsel2, n2)):
            for g in range(n // 16):
                sl = pl.ds(16 * g, 16)
                rows = 16 * g + iota16
                lanes = lax.bitwise_and(idx[sl], 127)
                bsel[sl] = plsc.load_gather(b128v, [rows, lanes])
        pltpu.sync_copy(rows1, wa_out.at[pl.ds(base1, n1)])
        pltpu.sync_copy(bsel1, ba_out.at[wid])
        pltpu.sync_copy(rows2, tw_out.at[pl.ds(base2, n2)])
        pltpu.sync_copy(bsel2, tb_out.at[wid])

    return gather(weight, bias128, aug_idx, label_idx)


def _tc_body(x_ref, w_ref, ba_ref, sf_ref, ids_ref, lab_ref,
             tw_ref, tb_ref, tf_ref, out_ref):
    x = x_ref[...]
    acc = lax.dot_general(x, w_ref[...], (((1,), (1,)), ((), ())),
                          preferred_element_type=jnp.float32)
    val = acc + (ba_ref[...] - jnp.log(sf_ref[...]))
    val = jnp.where(lab_ref[...] == ids_ref[...], _NEG_INF, val)
    t = (jnp.sum(x * tw_ref[...], axis=1, keepdims=True)
         + tb_ref[...] - jnp.log(tf_ref[...]))
    col0 = lax.broadcasted_iota(jnp.int32, val.shape, 1) == 0
    out_ref[...] = jnp.where(col0, t, val)


def kernel(inputs, labels, weight, bias, sample_ids, true_freq, sample_freq):
    b, d = inputs.shape
    ns = sample_ids.shape[0]
    v = weight.shape[0]
    nout = ns + 1

    labels_i = labels.astype(jnp.int32)
    sids_i = sample_ids.astype(jnp.int32)

    info = plsc.get_sparse_core_info()
    nw = info.num_cores * info.num_subcores
    step = max(16 * nw, 128)
    naug = ((nout + step - 1) // step) * step

    pad = naug - 1 - ns
    aug_idx = jnp.concatenate(
        [jnp.zeros((1,), jnp.int32), sids_i, jnp.zeros((pad,), jnp.int32)])
    aug_ids = jnp.concatenate(
        [jnp.full((1,), -1, jnp.int32), sids_i, jnp.full((pad,), -1, jnp.int32)]
    ).reshape(1, naug)
    aug_sf = jnp.concatenate(
        [jnp.ones((1,), jnp.float32), sample_freq, jnp.ones((pad,), jnp.float32)]
    ).reshape(1, naug)

    vb = ((v + 127) // 128) * 128
    bias128 = jnp.pad(bias, (0, vb - v)).reshape(vb // 128, 128)

    w_aug, b_aug, tw, tb = _sc_gather(
        weight, bias128, aug_idx.reshape(nw, naug // nw),
        labels_i.reshape(nw, b // nw), naug=naug)
    b_aug = b_aug.reshape(1, naug)
    tb = tb.reshape(b, 1)

    br = 256
    grid = (b // br,)
    out = pl.pallas_call(
        _tc_body,
        grid=grid,
        in_specs=[
            pl.BlockSpec((br, d), lambda i: (i, 0)),
            pl.BlockSpec((naug, d), lambda i: (0, 0)),
            pl.BlockSpec((1, naug), lambda i: (0, 0)),
            pl.BlockSpec((1, naug), lambda i: (0, 0)),
            pl.BlockSpec((1, naug), lambda i: (0, 0)),
            pl.BlockSpec((br, 1), lambda i: (i, 0)),
            pl.BlockSpec((br, d), lambda i: (i, 0)),
            pl.BlockSpec((br, 1), lambda i: (i, 0)),
            pl.BlockSpec((br, 1), lambda i: (i, 0)),
        ],
        out_specs=pl.BlockSpec((br, naug), lambda i: (i, 0)),
        out_shape=jax.ShapeDtypeStruct((b, nout), jnp.float32),
    )(inputs, w_aug, b_aug, aug_sf, aug_ids, labels_i.reshape(b, 1),
      tw, tb, true_freq.reshape(b, 1))
    return out


# speedup vs baseline: 1.6606x; 1.2066x over previous
import functools

import jax
import jax.numpy as jnp
from jax import lax
from jax.experimental import pallas as pl
from jax.experimental.pallas import tpu as pltpu
from jax.experimental.pallas import tpu_sc as plsc

_NEG_INF = float(-1e37)
_IDX_CHUNK = 128
_ROW_GRP = 16


def _chunks(n):
    out, off = [], 0
    while off < n:
        sz = min(_IDX_CHUNK, n - off)
        out.append((off, sz))
        off += sz
    return out


@functools.partial(jax.jit, static_argnames=("naug",))
def _sc_gather(weight, bias128, aug_idx, label_idx, *, naug):
    d = weight.shape[1]
    nw, n1 = aug_idx.shape
    n2 = label_idx.shape[1]
    b = nw * n2
    info = plsc.get_sparse_core_info()
    nc = info.num_cores
    ch1, ch2 = _chunks(n1), _chunks(n2)

    mesh = plsc.VectorSubcoreMesh(core_axis_name="c", subcore_axis_name="s")

    @functools.partial(
        pl.kernel,
        out_type=(
            jax.ShapeDtypeStruct((naug, d), jnp.float32),
            jax.ShapeDtypeStruct((nw, n1), jnp.float32),
            jax.ShapeDtypeStruct((b, d), jnp.float32),
            jax.ShapeDtypeStruct((nw, n2), jnp.float32),
        ),
        mesh=mesh,
        compiler_params=pltpu.CompilerParams(needs_layout_passes=False),
        scratch_types=(
            pltpu.VMEM((n1,), jnp.int32),
            pltpu.VMEM((n1,), jnp.int32),
            pltpu.VMEM((n1, d), jnp.float32),
            pltpu.VMEM((n1, 128), jnp.float32),
            pltpu.VMEM((n1,), jnp.float32),
            pltpu.VMEM((n2,), jnp.int32),
            pltpu.VMEM((n2,), jnp.int32),
            pltpu.VMEM((n2, d), jnp.float32),
            pltpu.VMEM((n2, 128), jnp.float32),
            pltpu.VMEM((n2,), jnp.float32),
            pltpu.SemaphoreType.DMA,
            pltpu.SemaphoreType.DMA,
        ),
    )
    def gather(w_hbm, b128_hbm, aidx_hbm, lidx_hbm,
               wa_out, ba_out, tw_out, tb_out,
               idx1, idx1b, rows1, b128_1, bsel1,
               idx2, idx2b, rows2, b128_2, bsel2, sem, rsem):
        iota16 = lax.iota(jnp.int32, 16)
        wid = lax.axis_index("s") * nc + lax.axis_index("c")
        base1 = wid * n1
        base2 = wid * n2
        pltpu.sync_copy(aidx_hbm.at[wid], idx1)
        pltpu.sync_copy(lidx_hbm.at[wid], idx2)
        for idx, idxb, n in ((idx1, idx1b, n1), (idx2, idx2b, n2)):
            for g in range(n // 16):
                sl = pl.ds(16 * g, 16)
                idxb[sl] = lax.shift_right_logical(idx[sl], 7)
        copies = []
        for off, sz in ch1:
            copies.append(pltpu.async_copy(
                b128_hbm.at[idx1b.at[pl.ds(off, sz)]], b128_1.at[pl.ds(off, sz)], sem))
        for off, sz in ch2:
            copies.append(pltpu.async_copy(
                b128_hbm.at[idx2b.at[pl.ds(off, sz)]], b128_2.at[pl.ds(off, sz)], sem))
        for idx, rows, n in ((idx1, rows1, n1), (idx2, rows2, n2)):
            ngrp = n // _ROW_GRP
            pend = []

            def issue(g, idx=idx, rows=rows):
                vec = idx[pl.ds(g * _ROW_GRP, 16)]
                cps = []
                for j in range(_ROW_GRP):
                    k = g * _ROW_GRP + j
                    cps.append(pltpu.async_copy(
                        w_hbm.at[pl.ds(vec[j], 1)], rows.at[pl.ds(k, 1)], rsem))
                return cps

            for g in range(ngrp):
                nxt = issue(g)
                for cp in pend:
                    cp.wait()
                pend = nxt
            for cp in pend:
                cp.wait()
        for c in copies:
            c.wait()
        for idx, b128v, bsel, n in ((idx1, b128_1, bsel1, n1),
                                    (idx2, b128_2, bsel2, n2)):
            for g in range(n // 16):
                sl = pl.ds(16 * g, 16)
                rows = 16 * g + iota16
                lanes = lax.bitwise_and(idx[sl], 127)
                bsel[sl] = plsc.load_gather(b128v, [rows, lanes])
        pltpu.sync_copy(rows1, wa_out.at[pl.ds(base1, n1)])
        pltpu.sync_copy(bsel1, ba_out.at[wid])
        pltpu.sync_copy(rows2, tw_out.at[pl.ds(base2, n2)])
        pltpu.sync_copy(bsel2, tb_out.at[wid])

    return gather(weight, bias128, aug_idx, label_idx)


def _tc_body(xt_ref, w_ref, ba_ref, sf_ref, ids_ref, lab_ref,
             tw_ref, tb_ref, tf_ref, out_ref, *, br):
    xt = xt_ref[...]
    acc = lax.dot_general(w_ref[...], xt, (((1,), (0,)), ((), ())),
                          preferred_element_type=jnp.float32)
    val = acc + (ba_ref[...] - jnp.log(sf_ref[...]))
    val = jnp.where(ids_ref[...] == lab_ref[...], _NEG_INF, val)
    tacc = lax.dot_general(tw_ref[...], xt, (((1,), (0,)), ((), ())),
                           preferred_element_type=jnp.float32)
    eye = (lax.broadcasted_iota(jnp.int32, (br, br), 0)
           == lax.broadcasted_iota(jnp.int32, (br, br), 1))
    t = (jnp.sum(jnp.where(eye, tacc, 0.0), axis=0, keepdims=True)
         + tb_ref[...] - jnp.log(tf_ref[...]))
    row0 = lax.broadcasted_iota(jnp.int32, val.shape, 0) == 0
    out_ref[...] = jnp.where(row0, t, val)


def kernel(inputs, labels, weight, bias, sample_ids, true_freq, sample_freq):
    b, d = inputs.shape
    ns = sample_ids.shape[0]
    v = weight.shape[0]
    nout = ns + 1

    labels_i = labels.astype(jnp.int32)
    sids_i = sample_ids.astype(jnp.int32)

    info = plsc.get_sparse_core_info()
    nw = info.num_cores * info.num_subcores
    step = max(16 * nw, 128)
    naug = ((nout + step - 1) // step) * step

    pad = naug - 1 - ns
    aug_idx = jnp.concatenate(
        [jnp.zeros((1,), jnp.int32), sids_i, jnp.zeros((pad,), jnp.int32)])
    aug_ids = jnp.concatenate(
        [jnp.full((1,), -1, jnp.int32), sids_i, jnp.full((pad,), -1, jnp.int32)]
    ).reshape(naug, 1)
    aug_sf = jnp.concatenate(
        [jnp.ones((1,), jnp.float32), sample_freq, jnp.ones((pad,), jnp.float32)]
    ).reshape(naug, 1)

    vb = ((v + 127) // 128) * 128
    bias128 = jnp.pad(bias, (0, vb - v)).reshape(vb // 128, 128)

    w_aug, b_aug, tw, tb = _sc_gather(
        weight, bias128, aug_idx.reshape(nw, naug // nw),
        labels_i.reshape(nw, b // nw), naug=naug)
    b_aug = b_aug.reshape(naug, 1)
    tb = tb.reshape(1, b)

    br = 256
    grid = (b // br,)
    out_t = pl.pallas_call(
        functools.partial(_tc_body, br=br),
        grid=grid,
        in_specs=[
            pl.BlockSpec((d, br), lambda i: (0, i)),
            pl.BlockSpec((naug, d), lambda i: (0, 0)),
            pl.BlockSpec((naug, 1), lambda i: (0, 0)),
            pl.BlockSpec((naug, 1), lambda i: (0, 0)),
            pl.BlockSpec((naug, 1), lambda i: (0, 0)),
            pl.BlockSpec((1, br), lambda i: (0, i)),
            pl.BlockSpec((br, d), lambda i: (i, 0)),
            pl.BlockSpec((1, br), lambda i: (0, i)),
            pl.BlockSpec((1, br), lambda i: (0, i)),
        ],
        out_specs=pl.BlockSpec((naug, br), lambda i: (0, i)),
        out_shape=jax.ShapeDtypeStruct((nout, b), jnp.float32),
    )(inputs.T, w_aug, b_aug, aug_sf, aug_ids, labels_i.reshape(1, b),
      tw, tb, true_freq.reshape(1, b))
    return out_t.T
